# baseline (speedup 1.0000x reference)
import functools

import jax
import jax.numpy as jnp
from jax import lax
from jax.experimental import pallas as pl
from jax.experimental.pallas import tpu as pltpu
from jax.experimental.pallas import tpu_sc as plsc

NC, NS, L = 2, 16, 16
NW = NC * NS
BATCH, SEQ, DIM = 16384, 200, 10
N = BATCH * SEQ
LB = SEQ // 8
BT = BATCH // 128
UNIT = 8192
UNITS = N // UNIT
U_MAX = -(-UNITS // NW)
TBL = 128

_mesh = plsc.VectorSubcoreMesh(core_axis_name="c", subcore_axis_name="s")


@functools.partial(
    pl.kernel,
    out_type=jax.ShapeDtypeStruct((SEQ, 1, BATCH), jnp.float32),
    mesh=_mesh,
    compiler_params=pltpu.CompilerParams(needs_layout_passes=False),
    scratch_types=[
        pltpu.VMEM((2 * DIM + 1, TBL), jnp.float32),
        pltpu.VMEM((TBL,), jnp.float32),
        pltpu.VMEM((4, 8, 8, 128), jnp.int32),
        pltpu.VMEM((4, 8, 1, 1024), jnp.float32),
        pltpu.SemaphoreType.DMA,
        pltpu.SemaphoreType.DMA,
        pltpu.SemaphoreType.DMA,
        pltpu.SemaphoreType.DMA,
        pltpu.SemaphoreType.DMA,
        pltpu.SemaphoreType.DMA,
        pltpu.SemaphoreType.DMA,
        pltpu.SemaphoreType.DMA,
    ],
)
def _embed_gather(ept_hbm, xp_hbm, out_hbm,
                  ept_v, table_v, ib, ob,
                  in_sem0, in_sem1, in_sem2, in_sem3,
                  out_sem0, out_sem1, out_sem2, out_sem3):
    in_sems = (in_sem0, in_sem1, in_sem2, in_sem3)
    out_sems = (out_sem0, out_sem1, out_sem2, out_sem3)
    wid = lax.axis_index("s") * NC + lax.axis_index("c")

    def unit_of(k):
        return wid + k * NW

    def in_copy(k, b):
        u = unit_of(k)
        row0 = (u >> 4) * 128 + (u & 15) * 8
        return pltpu.make_async_copy(
            xp_hbm.at[pl.ds(row0, 8), :, :], ib.at[b], in_sems[b])

    def out_copy(k, b):
        u = unit_of(k)
        return pltpu.make_async_copy(
            ob.at[b],
            out_hbm.at[pl.ds((u >> 4) * 8, 8), :,
                       pl.ds((u & 15) * 1024, 1024)],
            out_sems[b])

    for k in range(4):
        in_copy(k, k).start()

    pltpu.sync_copy(ept_hbm, ept_v)
    wbv = ept_v[2 * DIM, pl.ds(0, L)]
    for c in range(TBL // L):
        acc = jnp.zeros((L,), jnp.float32) + wbv[DIM]
        for d in range(DIM):
            acc = acc + (ept_v[d, pl.ds(c * L, L)]
                         + ept_v[DIM + d, pl.ds(c * L, L)]) * wbv[d]
        table_v[pl.ds(c * L, L)] = acc

    def step(k, b):
        in_copy(k, b).wait()

        @pl.when(k >= 4)
        def _():
            out_copy(k - 4, b).wait()

        @plsc.parallel_loop(0, UNIT, step=L, unroll=16)
        def _(j):
            iv = ib[b, j >> 10, (j >> 7) & 7, pl.ds(j & 127, L)]
            ob[b, (j >> 7) & 7, 0,
               pl.ds(((j >> 10) << 7) | (j & 127), L)] = plsc.load_gather(
                   table_v, [iv])

        out_copy(k, b).start()

        @pl.when(unit_of(k + 4) < UNITS)
        def _():
            in_copy(k + 4, b).start()

    def quad(g, carry):
        for b in range(4):
            k = g * 4 + b

            @pl.when(unit_of(k) < UNITS)
            def _():
                step(k, b)
        return carry

    lax.fori_loop(0, (U_MAX + 3) // 4, quad, 0)
    for b in range(4):
        last = U_MAX - 1 - ((U_MAX - 1 - b) % 4)

        @pl.when(unit_of(last) < UNITS)
        def _():
            out_copy(last, b).wait()

        if last >= 4:
            @pl.when(unit_of(last) >= UNITS)
            def _():
                out_copy(last - 4, b).wait()


def kernel(x, embed_in, embed_in_2, lin0_w, lin0_b):
    xp = (x.astype(jnp.int32)
           .reshape(BT, 128, LB, 8)
           .transpose(2, 0, 3, 1)
           .reshape(LB * BT, 8, 128))
    wrow = jnp.concatenate([lin0_w[0], lin0_b])[None, :]
    ept = jnp.pad(
        jnp.concatenate([embed_in.T, embed_in_2.T,
                         jnp.pad(wrow, ((0, 0), (0, 100 - DIM - 1)))], axis=0),
        ((0, 0), (0, TBL - 100)))
    out = _embed_gather(ept, xp)
    return out.transpose(2, 0, 1)


# speedup vs baseline: 1.0114x; 1.0114x over previous
import functools

import jax
import jax.numpy as jnp
from jax import lax
from jax.experimental import pallas as pl
from jax.experimental.pallas import tpu as pltpu
from jax.experimental.pallas import tpu_sc as plsc

NC, NS, L = 2, 16, 16
NW = NC * NS
BATCH, SEQ, DIM = 16384, 200, 10
N = BATCH * SEQ
LB = SEQ // 8
BT = BATCH // 128
UNIT = 8192
UNITS = N // UNIT
U_MAX = -(-UNITS // NW)
TBL = 128

_mesh = plsc.VectorSubcoreMesh(core_axis_name="c", subcore_axis_name="s")


@functools.partial(
    pl.kernel,
    out_type=jax.ShapeDtypeStruct((SEQ, 1, BATCH), jnp.float32),
    mesh=_mesh,
    compiler_params=pltpu.CompilerParams(needs_layout_passes=False),
    scratch_types=[
        pltpu.VMEM((2 * DIM + 1, TBL), jnp.float32),
        pltpu.VMEM((TBL,), jnp.float32),
        pltpu.VMEM((4, 8, 8, 128), jnp.int32),
        pltpu.VMEM((4, 8, 1, 1024), jnp.float32),
        pltpu.SemaphoreType.DMA,
        pltpu.SemaphoreType.DMA,
        pltpu.SemaphoreType.DMA,
        pltpu.SemaphoreType.DMA,
        pltpu.SemaphoreType.DMA,
        pltpu.SemaphoreType.DMA,
        pltpu.SemaphoreType.DMA,
        pltpu.SemaphoreType.DMA,
    ],
)
def _embed_gather(ept_hbm, xp_hbm, out_hbm,
                  ept_v, table_v, ib, ob,
                  in_sem0, in_sem1, in_sem2, in_sem3,
                  out_sem0, out_sem1, out_sem2, out_sem3):
    in_sems = (in_sem0, in_sem1, in_sem2, in_sem3)
    out_sems = (out_sem0, out_sem1, out_sem2, out_sem3)
    wid = lax.axis_index("s") * NC + lax.axis_index("c")

    def unit_of(k):
        return wid + k * NW

    def in_copy(k, b):
        u = unit_of(k)
        row0 = (u >> 4) * 128 + (u & 15) * 8
        return pltpu.make_async_copy(
            xp_hbm.at[pl.ds(row0, 8), :, :], ib.at[b], in_sems[b])

    def out_copy(k, b):
        u = unit_of(k)
        return pltpu.make_async_copy(
            ob.at[b],
            out_hbm.at[pl.ds((u >> 4) * 8, 8), :,
                       pl.ds((u & 15) * 1024, 1024)],
            out_sems[b])

    for k in range(4):
        in_copy(k, k).start()

    pltpu.sync_copy(ept_hbm, ept_v)
    wbv = ept_v[2 * DIM, pl.ds(0, L)]
    for c in range(TBL // L):
        acc = jnp.zeros((L,), jnp.float32) + wbv[DIM]
        for d in range(DIM):
            acc = acc + (ept_v[d, pl.ds(c * L, L)]
                         + ept_v[DIM + d, pl.ds(c * L, L)]) * wbv[d]
        table_v[pl.ds(c * L, L)] = acc

    def step(k, b):
        in_copy(k, b).wait()

        @pl.when(k >= 4)
        def _():
            out_copy(k - 4, b).wait()

        @plsc.parallel_loop(0, UNIT, step=L, unroll=8)
        def _(j):
            iv = ib[b, j >> 10, (j >> 7) & 7, pl.ds(j & 127, L)]
            ob[b, (j >> 7) & 7, 0,
               pl.ds(((j >> 10) << 7) | (j & 127), L)] = plsc.load_gather(
                   table_v, [iv])

        out_copy(k, b).start()

        @pl.when(unit_of(k + 4) < UNITS)
        def _():
            in_copy(k + 4, b).start()

    def quad(g, carry):
        for b in range(4):
            k = g * 4 + b

            @pl.when(unit_of(k) < UNITS)
            def _():
                step(k, b)
        return carry

    lax.fori_loop(0, (U_MAX + 3) // 4, quad, 0)
    for b in range(4):
        last = U_MAX - 1 - ((U_MAX - 1 - b) % 4)

        @pl.when(unit_of(last) < UNITS)
        def _():
            out_copy(last, b).wait()

        if last >= 4:
            @pl.when(unit_of(last) >= UNITS)
            def _():
                out_copy(last - 4, b).wait()


def kernel(x, embed_in, embed_in_2, lin0_w, lin0_b):
    xp = (x.astype(jnp.int32)
           .reshape(BT, 128, LB, 8)
           .transpose(2, 0, 3, 1)
           .reshape(LB * BT, 8, 128))
    wrow = jnp.concatenate([lin0_w[0], lin0_b])[None, :]
    ept = jnp.pad(
        jnp.concatenate([embed_in.T, embed_in_2.T,
                         jnp.pad(wrow, ((0, 0), (0, 100 - DIM - 1)))], axis=0),
        ((0, 0), (0, TBL - 100)))
    out = _embed_gather(ept, xp)
    return out.transpose(2, 0, 1)
